# initial kernel scaffold (unmeasured)
import functools

import jax
import jax.numpy as jnp
from jax import lax
from jax.experimental import pallas as pl
from jax.experimental.pallas import tpu as pltpu

N_DEV = 32
COMM_DTYPE = jnp.float32


def kernel(x, w_mat):
    m, k_local = x.shape
    _, n = w_mat.shape
    chunk = m // N_DEV
    n_hops = 2 * (N_DEV - 1)

    def body(x_ref, w_ref, out_ref, comm_ref, send_sems, recv_sems, credit_sem):
        my = lax.axis_index("i")
        left = lax.rem(my - 1 + N_DEV, N_DEV)
        right = lax.rem(my + 1, N_DEV)

        barrier_sem = pltpu.get_barrier_semaphore()
        for nbr in (left, right):
            pl.semaphore_signal(
                barrier_sem, inc=1,
                device_id=(nbr,), device_id_type=pl.DeviceIdType.MESH,
            )
        pl.semaphore_wait(barrier_sem, 2)

        out_ref[...] = jnp.dot(
            x_ref[...], w_ref[...], preferred_element_type=jnp.float32
        )

        comm_ref[0] = out_ref[pl.ds(my * chunk, chunk), :].astype(COMM_DTYPE)

        for j in range(n_hops):
            s = j % 2
            r = (j + 1) % 2
            if j >= 1:
                pl.semaphore_wait(credit_sem, 1)
            rdma = pltpu.make_async_remote_copy(
                src_ref=comm_ref.at[s],
                dst_ref=comm_ref.at[r],
                send_sem=send_sems.at[s],
                recv_sem=recv_sems.at[r],
                device_id=(right,),
                device_id_type=pl.DeviceIdType.MESH,
            )
            rdma.start()
            rdma.wait()
            if j < N_DEV - 1:
                c = lax.rem(my - j - 1 + N_DEV, N_DEV)
                acc = comm_ref[r].astype(jnp.float32) + out_ref[
                    pl.ds(c * chunk, chunk), :
                ]
                val = acc.astype(COMM_DTYPE)
                comm_ref[r] = val
                if j == N_DEV - 2:
                    out_ref[pl.ds(c * chunk, chunk), :] = val.astype(jnp.float32)
            else:
                g = j - (N_DEV - 1)
                c = lax.rem(my - g + N_DEV, N_DEV)
                out_ref[pl.ds(c * chunk, chunk), :] = comm_ref[r].astype(
                    jnp.float32
                )
            if j < n_hops - 1:
                pl.semaphore_signal(
                    credit_sem, inc=1,
                    device_id=(left,), device_id_type=pl.DeviceIdType.MESH,
                )

        y = out_ref[...]
        amax = jnp.max(jnp.abs(y))
        scale = amax / 127.0
        q = jnp.clip(jnp.round(y / scale), -127.0, 127.0)
        out_ref[...] = q * scale

        @functools.partial(
            pl.run_scoped, second_barrier=pltpu.SemaphoreType.REGULAR
        )
        def _(second_barrier):
            for nbr in (left, right):
                pl.semaphore_signal(
                    second_barrier, inc=1,
                    device_id=(nbr,), device_id_type=pl.DeviceIdType.MESH,
                )
            pl.semaphore_wait(second_barrier, 2)

    return pl.pallas_call(
        body,
        out_shape=jax.ShapeDtypeStruct((m, n), jnp.float32),
        in_specs=[
            pl.BlockSpec(memory_space=pltpu.VMEM),
            pl.BlockSpec(memory_space=pltpu.VMEM),
        ],
        out_specs=pl.BlockSpec(memory_space=pltpu.VMEM),
        scratch_shapes=[
            pltpu.VMEM((2, chunk, n), COMM_DTYPE),
            pltpu.SemaphoreType.DMA((2,)),
            pltpu.SemaphoreType.DMA((2,)),
            pltpu.SemaphoreType.REGULAR,
        ],
        compiler_params=pltpu.CompilerParams(collective_id=0),
    )(x, w_mat)


# baseline (device time: 1187265 ns/iter reference)
import functools
import os

import jax
import jax.numpy as jnp
from jax import lax
from jax.experimental import pallas as pl
from jax.experimental.pallas import tpu as pltpu

N_DEV = 32
COMM_DTYPE = jnp.float32


def kernel(x, w_mat):
    m, k_local = x.shape
    _, n = w_mat.shape
    chunk = m // N_DEV
    n_hops = 2 * (N_DEV - 1)
    n_hops = int(os.environ.get("DEBUG_HOPS", n_hops))

    def body(x_ref, w_ref, out_ref, comm_ref, send_sems, recv_sems, credit_sem):
        my = lax.axis_index("i")
        left = lax.rem(my - 1 + N_DEV, N_DEV)
        right = lax.rem(my + 1, N_DEV)

        barrier_sem = pltpu.get_barrier_semaphore()
        for nbr in (left, right):
            pl.semaphore_signal(
                barrier_sem, inc=1,
                device_id=(nbr,), device_id_type=pl.DeviceIdType.MESH,
            )
        pl.semaphore_wait(barrier_sem, 2)

        wb = w_ref[...].astype(jnp.bfloat16)
        gemm_rows = 512
        for b in range(m // gemm_rows):
            xb = x_ref[pl.ds(b * gemm_rows, gemm_rows), :].astype(jnp.bfloat16)
            out_ref[pl.ds(b * gemm_rows, gemm_rows), :] = jnp.dot(
                xb, wb, preferred_element_type=jnp.float32
            )

        comm_ref[0] = out_ref[pl.ds(my * chunk, chunk), :].astype(COMM_DTYPE)

        for j in range(n_hops):
            s = j % 2
            r = (j + 1) % 2
            if j >= 1:
                pl.semaphore_wait(credit_sem, 1)
            rdma = pltpu.make_async_remote_copy(
                src_ref=comm_ref.at[s],
                dst_ref=comm_ref.at[r],
                send_sem=send_sems.at[s],
                recv_sem=recv_sems.at[r],
                device_id=(right,),
                device_id_type=pl.DeviceIdType.MESH,
            )
            rdma.start()
            rdma.wait()
            if j < N_DEV - 1:
                c = lax.rem(my - j - 1 + N_DEV, N_DEV)
                acc = comm_ref[r].astype(jnp.float32) + out_ref[
                    pl.ds(c * chunk, chunk), :
                ]
                val = acc.astype(COMM_DTYPE)
                comm_ref[r] = val
                if j == N_DEV - 2:
                    out_ref[pl.ds(c * chunk, chunk), :] = val.astype(jnp.float32)
            else:
                g = j - (N_DEV - 1)
                c = lax.rem(my - g + N_DEV, N_DEV)
                out_ref[pl.ds(c * chunk, chunk), :] = comm_ref[r].astype(
                    jnp.float32
                )
            if j < n_hops - 1:
                pl.semaphore_signal(
                    credit_sem, inc=1,
                    device_id=(left,), device_id_type=pl.DeviceIdType.MESH,
                )

        ep_rows = 512
        amax = jnp.float32(0.0)
        for b in range(m // ep_rows):
            blk = out_ref[pl.ds(b * ep_rows, ep_rows), :]
            amax = jnp.maximum(amax, jnp.max(jnp.abs(blk)))
        scale = amax / 127.0
        for b in range(m // ep_rows):
            blk = out_ref[pl.ds(b * ep_rows, ep_rows), :]
            q = jnp.clip(jnp.round(blk / scale), -127.0, 127.0)
            out_ref[pl.ds(b * ep_rows, ep_rows), :] = q * scale

        @functools.partial(
            pl.run_scoped, second_barrier=pltpu.SemaphoreType.REGULAR
        )
        def _(second_barrier):
            for nbr in (left, right):
                pl.semaphore_signal(
                    second_barrier, inc=1,
                    device_id=(nbr,), device_id_type=pl.DeviceIdType.MESH,
                )
            pl.semaphore_wait(second_barrier, 2)

    return pl.pallas_call(
        body,
        out_shape=jax.ShapeDtypeStruct((m, n), jnp.float32),
        in_specs=[
            pl.BlockSpec(memory_space=pltpu.VMEM),
            pl.BlockSpec(memory_space=pltpu.VMEM),
        ],
        out_specs=pl.BlockSpec(memory_space=pltpu.VMEM),
        scratch_shapes=[
            pltpu.VMEM((2, chunk, n), COMM_DTYPE),
            pltpu.SemaphoreType.DMA((2,)),
            pltpu.SemaphoreType.DMA((2,)),
            pltpu.SemaphoreType.REGULAR,
        ],
        compiler_params=pltpu.CompilerParams(
            collective_id=0, vmem_limit_bytes=60 * 1024 * 1024
        ),
    )(x, w_mat)
